# baseline (speedup 1.0000x reference)
import functools

import jax
import jax.numpy as jnp
from jax import lax
from jax.experimental import pallas as pl
from jax.experimental.pallas import tpu as pltpu
from jax.experimental.pallas import tpu_sc as plsc

_OUT_SHAPE = (16384, 26, 8)
_W = 16384 * 26 * 8
_NC, _NS = 2, 16
_NW = _NC * _NS
_PER = _W // _NW
_BUF_W = 8192
_NDMA = _PER // _BUF_W


def _sc_broadcast(table16):
    mesh = plsc.VectorSubcoreMesh(core_axis_name="c", subcore_axis_name="s")

    @functools.partial(
        pl.kernel,
        mesh=mesh,
        out_type=jax.ShapeDtypeStruct((_W,), jnp.float32),
        scratch_types=[
            pltpu.VMEM((16,), jnp.float32),
            pltpu.VMEM((_BUF_W,), jnp.float32),
            pltpu.SemaphoreType.DMA,
        ],
    )
    def k(tbl_hbm, out_hbm, tbl_v, buf, sem):
        wid = lax.axis_index("s") * _NC + lax.axis_index("c")
        base = wid * _PER
        pltpu.sync_copy(tbl_hbm, tbl_v)
        v = tbl_v[...]

        def fill(j, carry):
            buf[pl.ds(j * 16, 16)] = v
            return carry

        lax.fori_loop(0, _BUF_W // 16, fill, 0)
        for d in range(_NDMA):
            pltpu.make_async_copy(
                buf, out_hbm.at[pl.ds(base + d * _BUF_W, _BUF_W)], sem
            ).start()
        for d in range(_NDMA):
            pltpu.make_async_copy(
                buf, out_hbm.at[pl.ds(base + d * _BUF_W, _BUF_W)], sem
            ).wait()

    return k(table16)


def kernel(input, table):
    del input
    table16 = jnp.tile(table.reshape(-1), 2)
    return _sc_broadcast(table16).reshape(_OUT_SHAPE)


# speedup vs baseline: 9.4703x; 9.4703x over previous
import functools

import jax
import jax.numpy as jnp
from jax import lax
from jax.experimental import pallas as pl
from jax.experimental.pallas import tpu as pltpu
from jax.experimental.pallas import tpu_sc as plsc

_B, _C, _D = 16384, 26, 8
_R = _C * _D
_NC, _NS = 2, 16
_NW = _NC * _NS
_COLS = 1024
_NCOL = _B // _COLS
_NCHUNK = (_R // 8) * _NCOL
_PER = _NCHUNK // _NW


def _sc_broadcast(pattern):
    mesh = plsc.VectorSubcoreMesh(core_axis_name="c", subcore_axis_name="s")

    @functools.partial(
        pl.kernel,
        mesh=mesh,
        out_type=jax.ShapeDtypeStruct((_R, _B), jnp.float32),
        scratch_types=[
            pltpu.VMEM((8, _COLS), jnp.float32),
            pltpu.SemaphoreType.DMA,
        ],
    )
    def k(pat_hbm, out_hbm, buf, sem):
        wid = lax.axis_index("s") * _NC + lax.axis_index("c")
        u0 = wid * _PER
        pltpu.sync_copy(pat_hbm, buf)
        def _chunk(i):
            u = u0 + i
            s = u // _NCOL
            c = lax.rem(u, _NCOL)
            return out_hbm.at[pl.ds(s * 8, 8), pl.ds(c * _COLS, _COLS)]

        for i in range(_PER):
            pltpu.make_async_copy(buf, _chunk(i), sem).start()
        for i in range(_PER):
            pltpu.make_async_copy(buf, _chunk(i), sem).wait()

    return k(pattern)


def kernel(input, table):
    del input
    pattern = jnp.broadcast_to(table[0][:, None], (8, _COLS))
    out2d = _sc_broadcast(pattern)
    return out2d.reshape(_C, _D, _B).transpose(2, 0, 1)


# speedup vs baseline: 23.7618x; 2.5091x over previous
import jax
import jax.numpy as jnp
from jax.experimental import pallas as pl

_B, _C, _D = 16384, 26, 8
_R = _C * _D
_BLK = 1024
_GRID = _B // _BLK


def _body(pat_ref, out_ref):
    out_ref[...] = jnp.broadcast_to(pat_ref[...], (_R, _BLK))


def _tc_broadcast(pat):
    return pl.pallas_call(
        _body,
        grid=(_GRID,),
        in_specs=[pl.BlockSpec((_R, 1), lambda i: (0, 0))],
        out_specs=pl.BlockSpec((_R, _BLK), lambda i: (0, i)),
        out_shape=jax.ShapeDtypeStruct((_R, _B), jnp.float32),
    )(pat)


def kernel(input, table):
    del input
    pat = jnp.tile(table.reshape(-1), _C)[:, None]
    out2d = _tc_broadcast(pat)
    return out2d.reshape(_C, _D, _B).transpose(2, 0, 1)
